# baseline (device time: 92265 ns/iter reference)
import jax
import jax.numpy as jnp
from jax import lax
from jax.experimental import pallas as pl
from jax.experimental.pallas import tpu as pltpu

N_DEV = 32


def _full_barrier(me):
    barrier_sem = pltpu.get_barrier_semaphore()
    for k in range(1, N_DEV):
        peer = lax.rem(me + k, N_DEV)
        pl.semaphore_signal(
            barrier_sem, inc=1,
            device_id=(peer,), device_id_type=pl.DeviceIdType.MESH,
        )
    pl.semaphore_wait(barrier_sem, N_DEV - 1)


def _a2a(x_shard):
    k_glob, m_blk = x_shard.shape

    def body(x_ref, out_ref, send_sems, recv_sems):
        me = lax.axis_index("i")
        _full_barrier(me)

        out_ref[:, pl.ds(me * m_blk, m_blk)] = x_ref[pl.ds(me * m_blk, m_blk), :]

        rdmas = []
        for k in range(1, N_DEV):
            tgt = lax.rem(me + k, N_DEV)
            rdma = pltpu.make_async_remote_copy(
                src_ref=x_ref.at[pl.ds(tgt * m_blk, m_blk), :],
                dst_ref=out_ref.at[:, pl.ds(me * m_blk, m_blk)],
                send_sem=send_sems.at[k - 1],
                recv_sem=recv_sems.at[k - 1],
                device_id=(tgt,),
                device_id_type=pl.DeviceIdType.MESH,
            )
            rdma.start()
            rdmas.append(rdma)
        for rdma in rdmas:
            rdma.wait()

    return pl.pallas_call(
        body,
        out_shape=jax.ShapeDtypeStruct((m_blk, k_glob), x_shard.dtype),
        in_specs=[pl.BlockSpec(memory_space=pltpu.VMEM)],
        out_specs=pl.BlockSpec(memory_space=pltpu.VMEM),
        scratch_shapes=[
            pltpu.SemaphoreType.DMA((N_DEV - 1,)),
            pltpu.SemaphoreType.DMA((N_DEV - 1,)),
        ],
        compiler_params=pltpu.CompilerParams(collective_id=0),
    )(x_shard)


def _gemm_relu(xt, w_mat):
    m_blk, k_glob = xt.shape
    _, n_glob = w_mat.shape
    n_blk = 512
    grid = (n_glob // n_blk,)

    def body(x_ref, w_ref, y_ref):
        y_ref[...] = jnp.maximum(
            jnp.dot(x_ref[...], w_ref[...], preferred_element_type=jnp.float32),
            0.0,
        )

    return pl.pallas_call(
        body,
        grid=grid,
        in_specs=[
            pl.BlockSpec((m_blk, k_glob), lambda j: (0, 0)),
            pl.BlockSpec((k_glob, n_blk), lambda j: (0, j)),
        ],
        out_specs=pl.BlockSpec((m_blk, n_blk), lambda j: (0, j)),
        out_shape=jax.ShapeDtypeStruct((m_blk, n_glob), jnp.float32),
    )(xt, w_mat)


def _global_quant(y):
    m_blk, n_glob = y.shape

    def body(y_ref, out_ref, amax_buf, send_sems, recv_sems):
        me = lax.axis_index("i")

        local_amax = jnp.max(y_ref[...])
        amax_buf[0, :, :] = jnp.full((8, 128), local_amax, jnp.float32)

        _full_barrier(me)

        rdmas = []
        for k in range(1, N_DEV):
            tgt = lax.rem(me + k, N_DEV)
            rdma = pltpu.make_async_remote_copy(
                src_ref=amax_buf.at[0],
                dst_ref=amax_buf.at[k],
                send_sem=send_sems.at[k - 1],
                recv_sem=recv_sems.at[k - 1],
                device_id=(tgt,),
                device_id_type=pl.DeviceIdType.MESH,
            )
            rdma.start()
            rdmas.append(rdma)
        for rdma in rdmas:
            rdma.wait()

        gmax = jnp.max(amax_buf[...])
        scale = jnp.maximum(gmax, 1e-30) / 448.0
        q = jnp.minimum(y_ref[...] / scale, 448.0)
        q = q.astype(jnp.float8_e4m3fn).astype(jnp.float32)
        out_ref[...] = q * scale

    return pl.pallas_call(
        body,
        out_shape=jax.ShapeDtypeStruct((m_blk, n_glob), jnp.float32),
        in_specs=[pl.BlockSpec(memory_space=pltpu.VMEM)],
        out_specs=pl.BlockSpec(memory_space=pltpu.VMEM),
        scratch_shapes=[
            pltpu.VMEM((N_DEV, 8, 128), jnp.float32),
            pltpu.SemaphoreType.DMA((N_DEV - 1,)),
            pltpu.SemaphoreType.DMA((N_DEV - 1,)),
        ],
        compiler_params=pltpu.CompilerParams(collective_id=1),
    )(y)


def kernel(x, w_mat):
    xt = _a2a(x)
    y = _gemm_relu(xt, w_mat)
    return _global_quant(y)
